# Initial kernel scaffold; baseline (speedup 1.0000x reference)
import jax
import jax.numpy as jnp
from jax.experimental import pallas as pl


def kernel(x, edge_index, batch, emb_W, emb_b, conv_W, conv_b, bn_g, bn_b, mlp_W1, mlp_b1, mlp_bn_g, mlp_bn_b, mlp_W2, mlp_b2):
    raise NotImplementedError("write your pallas kernel here")


# speedup vs baseline: 13.7937x; 13.7937x over previous
import functools

import jax
import jax.numpy as jnp
from jax import lax
from jax.experimental import pallas as pl
from jax.experimental.pallas import tpu as pltpu
from jax.experimental.pallas import tpu_sc as plsc

_EPS = 1e-5
_G = 500
_GP = 512
_NC = 2
_NS = 16
_C = 128
_ZR = 80


def _mesh():
    return plsc.VectorSubcoreMesh(core_axis_name="c", subcore_axis_name="s")


def _sc_degree(dst, n):
    e = dst.shape[0]
    chunks = e // _C
    per_core = chunks // _NC
    row_chunks = n // _ZR

    @functools.partial(
        pl.kernel,
        out_type=jax.ShapeDtypeStruct((_NC, n, 16), jnp.float32),
        mesh=_mesh(),
        scratch_types=[
            pltpu.VMEM((1, _C), jnp.int32),
            pltpu.VMEM((_C, 16), jnp.float32),
            pltpu.VMEM((_ZR, 16), jnp.float32),
            pltpu.VMEM_SHARED((n, 16), jnp.float32),
        ],
    )
    def k(dst_hbm, out_hbm, didx, ones_v, ztile, acc):
        cid = lax.axis_index("c")
        sid = lax.axis_index("s")

        one16 = jnp.ones((16,), jnp.float32)
        zero16 = jnp.zeros((16,), jnp.float32)

        @pl.loop(0, _C)
        def _(i):
            ones_v[i] = one16

        @pl.loop(0, _ZR)
        def _(i):
            ztile[i] = zero16

        @pl.loop(sid, row_chunks, step=_NS)
        def _(t):
            pltpu.sync_copy(ztile, acc.at[pl.ds(t * _ZR, _ZR)])

        plsc.subcore_barrier()

        @pl.loop(cid * per_core + sid, (cid + 1) * per_core, step=_NS)
        def _(t):
            pltpu.sync_copy(dst_hbm.at[pl.ds(t * _C, _C)], didx.at[0])
            pltpu.sync_copy(ones_v, acc.at[didx.at[0]], add=True)

        plsc.subcore_barrier()

        @pl.loop(sid, row_chunks, step=_NS)
        def _(t):
            pltpu.sync_copy(acc.at[pl.ds(t * _ZR, _ZR)],
                            out_hbm.at[cid].at[pl.ds(t * _ZR, _ZR)])

    return k(dst)


def _sc_scatter(u, src, dst):
    n, d = u.shape
    e = src.shape[0]
    chunks = e // _C
    per_core = chunks // _NC
    row_chunks = n // _ZR

    @functools.partial(
        pl.kernel,
        out_type=jax.ShapeDtypeStruct((_NC, n, d), jnp.float32),
        mesh=_mesh(),
        scratch_types=[
            pltpu.VMEM((1, _C), jnp.int32),
            pltpu.VMEM((1, _C), jnp.int32),
            pltpu.VMEM((_C, d), jnp.float32),
            pltpu.VMEM((_ZR, d), jnp.float32),
            pltpu.VMEM_SHARED((n, d), jnp.float32),
            pltpu.SemaphoreType.DMA,
        ],
    )
    def k(u_hbm, src_hbm, dst_hbm, out_hbm, sidx, didx, rows, ztile, acc, sem):
        cid = lax.axis_index("c")
        sid = lax.axis_index("s")

        zero16 = jnp.zeros((16,), jnp.float32)

        @pl.loop(0, _ZR)
        def _(i):
            @pl.loop(0, d, step=16)
            def _(j):
                ztile[i, pl.ds(j, 16)] = zero16

        @pl.loop(sid, row_chunks, step=_NS)
        def _(t):
            pltpu.sync_copy(ztile, acc.at[pl.ds(t * _ZR, _ZR)])

        plsc.subcore_barrier()

        @pl.loop(cid * per_core + sid, (cid + 1) * per_core, step=_NS)
        def _(t):
            pltpu.sync_copy(src_hbm.at[pl.ds(t * _C, _C)], sidx.at[0])
            pltpu.sync_copy(dst_hbm.at[pl.ds(t * _C, _C)], didx.at[0])
            pltpu.async_copy(u_hbm.at[sidx.at[0]], rows, sem).wait()
            pltpu.sync_copy(rows, acc.at[didx.at[0]], add=True)

        plsc.subcore_barrier()

        @pl.loop(sid, row_chunks, step=_NS)
        def _(t):
            pltpu.sync_copy(acc.at[pl.ds(t * _ZR, _ZR)],
                            out_hbm.at[cid].at[pl.ds(t * _ZR, _ZR)])

    return k(u, src, dst)


def _sc_pool(h, batch):
    n, d = h.shape
    row_chunks = n // _ZR
    g_chunks = 8
    gr = _GP // g_chunks

    @functools.partial(
        pl.kernel,
        out_type=(jax.ShapeDtypeStruct((_NC, _GP, d), jnp.float32),
                  jax.ShapeDtypeStruct((_NC, _GP, 16), jnp.float32)),
        mesh=_mesh(),
        scratch_types=[
            pltpu.VMEM((1, _ZR), jnp.int32),
            pltpu.VMEM((_ZR, d), jnp.float32),
            pltpu.VMEM((_ZR, 16), jnp.float32),
            pltpu.VMEM((gr, d), jnp.float32),
            pltpu.VMEM((gr, 16), jnp.float32),
            pltpu.VMEM_SHARED((_GP, d), jnp.float32),
            pltpu.VMEM_SHARED((_GP, 16), jnp.float32),
        ],
    )
    def k(h_hbm, b_hbm, osum_hbm, ocnt_hbm,
          bidx, hrows, ones_v, zs, zc, acc_s, acc_c):
        cid = lax.axis_index("c")
        sid = lax.axis_index("s")
        wid = cid * _NS + sid

        one16 = jnp.ones((16,), jnp.float32)
        zero16 = jnp.zeros((16,), jnp.float32)

        @pl.loop(0, _ZR)
        def _(i):
            ones_v[i] = one16

        @pl.loop(0, gr)
        def _(i):
            zc[i] = zero16

            @pl.loop(0, d, step=16)
            def _(j):
                zs[i, pl.ds(j, 16)] = zero16

        @pl.when(sid < g_chunks)
        def _():
            pltpu.sync_copy(zs, acc_s.at[pl.ds(sid * gr, gr)])
            pltpu.sync_copy(zc, acc_c.at[pl.ds(sid * gr, gr)])

        plsc.subcore_barrier()

        @pl.loop(wid, row_chunks, step=_NC * _NS)
        def _(t):
            pltpu.sync_copy(b_hbm.at[pl.ds(t * _ZR, _ZR)], bidx.at[0])
            pltpu.sync_copy(h_hbm.at[pl.ds(t * _ZR, _ZR)], hrows)
            pltpu.sync_copy(hrows, acc_s.at[bidx.at[0]], add=True)
            pltpu.sync_copy(ones_v, acc_c.at[bidx.at[0]], add=True)

        plsc.subcore_barrier()

        @pl.when(sid < g_chunks)
        def _():
            pltpu.sync_copy(acc_s.at[pl.ds(sid * gr, gr)],
                            osum_hbm.at[cid].at[pl.ds(sid * gr, gr)])
            pltpu.sync_copy(acc_c.at[pl.ds(sid * gr, gr)],
                            ocnt_hbm.at[cid].at[pl.ds(sid * gr, gr)])

    return k(h, batch)


_BLK = 1000


def _dot(a, b):
    return jnp.dot(a, b, preferred_element_type=jnp.float32,
                   precision=lax.Precision.HIGHEST)


def _tc_embed(x, emb_W, emb_b, cw0):
    n, d = x.shape

    def body(x_ref, w_ref, b_ref, cw_ref, o_ref):
        h = _dot(x_ref[...], w_ref[...]) + b_ref[...]
        o_ref[...] = _dot(h, cw_ref[...])

    return pl.pallas_call(
        body,
        grid=(n // _BLK,),
        in_specs=[
            pl.BlockSpec((_BLK, d), lambda i: (i, 0)),
            pl.BlockSpec((d, d), lambda i: (0, 0)),
            pl.BlockSpec((1, d), lambda i: (0, 0)),
            pl.BlockSpec((d, d), lambda i: (0, 0)),
        ],
        out_specs=pl.BlockSpec((_BLK, d), lambda i: (i, 0)),
        out_shape=jax.ShapeDtypeStruct((n, d), jnp.float32),
    )(x, emb_W, emb_b.reshape(1, d), cw0)


def _dinv_from(degp_ref):
    deg = degp_ref[0, :, 0] + degp_ref[1, :, 0] + 1.0
    return lax.rsqrt(deg)[:, None]


def _tc_scale(m0, degp):
    n, d = m0.shape

    def body(m_ref, g_ref, o_ref):
        o_ref[...] = _dinv_from(g_ref) * m_ref[...]

    return pl.pallas_call(
        body,
        grid=(n // _BLK,),
        in_specs=[
            pl.BlockSpec((_BLK, d), lambda i: (i, 0)),
            pl.BlockSpec((2, _BLK, 16), lambda i: (0, i, 0)),
        ],
        out_specs=pl.BlockSpec((_BLK, d), lambda i: (i, 0)),
        out_shape=jax.ShapeDtypeStruct((n, d), jnp.float32),
    )(m0, degp)


def _bn_relu(agg, g_ref, b_ref):
    scale = g_ref[...] * (1.0 / jnp.sqrt(1.0 + _EPS))
    return jnp.maximum(scale * agg + b_ref[...], 0.0)


def _tc_layer(p, u, degp, bn_g, bn_b, conv_b, cw_next):
    n, d = u.shape

    def body(p_ref, u_ref, g_ref, bg_ref, bb_ref, cb_ref, cw_ref, o_ref):
        dinv = _dinv_from(g_ref)
        agg = dinv * (p_ref[0] + p_ref[1] + u_ref[...]) + cb_ref[...]
        h = _bn_relu(agg, bg_ref, bb_ref)
        o_ref[...] = dinv * _dot(h, cw_ref[...])

    return pl.pallas_call(
        body,
        grid=(n // _BLK,),
        in_specs=[
            pl.BlockSpec((2, _BLK, d), lambda i: (0, i, 0)),
            pl.BlockSpec((_BLK, d), lambda i: (i, 0)),
            pl.BlockSpec((2, _BLK, 16), lambda i: (0, i, 0)),
            pl.BlockSpec((1, d), lambda i: (0, 0)),
            pl.BlockSpec((1, d), lambda i: (0, 0)),
            pl.BlockSpec((1, d), lambda i: (0, 0)),
            pl.BlockSpec((d, d), lambda i: (0, 0)),
        ],
        out_specs=pl.BlockSpec((_BLK, d), lambda i: (i, 0)),
        out_shape=jax.ShapeDtypeStruct((n, d), jnp.float32),
    )(p, u, degp, bn_g.reshape(1, d), bn_b.reshape(1, d),
      conv_b.reshape(1, d), cw_next)


def _tc_last(p, u, degp, bn_g, bn_b, conv_b):
    n, d = u.shape

    def body(p_ref, u_ref, g_ref, bg_ref, bb_ref, cb_ref, o_ref):
        dinv = _dinv_from(g_ref)
        agg = dinv * (p_ref[0] + p_ref[1] + u_ref[...]) + cb_ref[...]
        o_ref[...] = _bn_relu(agg, bg_ref, bb_ref)

    return pl.pallas_call(
        body,
        grid=(n // _BLK,),
        in_specs=[
            pl.BlockSpec((2, _BLK, d), lambda i: (0, i, 0)),
            pl.BlockSpec((_BLK, d), lambda i: (i, 0)),
            pl.BlockSpec((2, _BLK, 16), lambda i: (0, i, 0)),
            pl.BlockSpec((1, d), lambda i: (0, 0)),
            pl.BlockSpec((1, d), lambda i: (0, 0)),
            pl.BlockSpec((1, d), lambda i: (0, 0)),
        ],
        out_specs=pl.BlockSpec((_BLK, d), lambda i: (i, 0)),
        out_shape=jax.ShapeDtypeStruct((n, d), jnp.float32),
    )(p, u, degp, bn_g.reshape(1, d), bn_b.reshape(1, d), conv_b.reshape(1, d))


def _tc_head(sums, cnts, mlp_W1, mlp_b1, mlp_bn_g, mlp_bn_b, mlp_W2, mlp_b2):
    g, d = sums.shape[1], sums.shape[2]

    def body(s_ref, c_ref, w1_ref, b1_ref, g_ref, b_ref, w2_ref, b2_ref,
             o_ref):
        cnt = c_ref[0, :, 0] + c_ref[1, :, 0]
        pooled = (s_ref[0] + s_ref[1]) / jnp.maximum(cnt, 1.0)[:, None]
        t = _dot(pooled, w1_ref[...]) + b1_ref[...]
        h2 = _bn_relu(t, g_ref, b_ref)
        o_ref[...] = jnp.sum(h2 * w2_ref[...], axis=1,
                             keepdims=True) + b2_ref[...]

    out = pl.pallas_call(
        body,
        out_shape=jax.ShapeDtypeStruct((g, 1), jnp.float32),
    )(sums, cnts, mlp_W1, mlp_b1.reshape(1, d), mlp_bn_g.reshape(1, d),
      mlp_bn_b.reshape(1, d), mlp_W2.reshape(1, d), mlp_b2.reshape(1, 1))
    return out[:_G]


def kernel(x, edge_index, batch, emb_W, emb_b, conv_W, conv_b, bn_g, bn_b,
           mlp_W1, mlp_b1, mlp_bn_g, mlp_bn_b, mlp_W2, mlp_b2):
    n, d = x.shape
    num_layers = conv_W.shape[0]
    src = edge_index[0]
    dst = edge_index[1]

    degp = _sc_degree(dst, n)
    m0 = _tc_embed(x, emb_W, emb_b, conv_W[0])
    u = _tc_scale(m0, degp)
    h = None
    for l in range(num_layers):
        p = _sc_scatter(u, src, dst)
        if l + 1 < num_layers:
            u = _tc_layer(p, u, degp, bn_g[l], bn_b[l], conv_b[l],
                          conv_W[l + 1])
        else:
            h = _tc_last(p, u, degp, bn_g[l], bn_b[l], conv_b[l])
    sums, cnts = _sc_pool(h, batch)
    return _tc_head(sums, cnts, mlp_W1, mlp_b1, mlp_bn_g, mlp_bn_b,
                    mlp_W2, mlp_b2)
